# baseline (speedup 1.0000x reference)
import jax
import jax.numpy as jnp
from jax import lax
from jax.experimental import pallas as pl
from jax.experimental.pallas import tpu as pltpu
from jax.experimental.pallas import tpu_sc as plsc

R = 128
N = 10000
E = 320000
NC = 2
NS = 16
NW = NC * NS
EW = E // NW
L = 16

_HI = -65536


def _body(mp_hbm, idx_hbm, out_hbm,
          ipk, rowA, rowB, obA, obB,
          semA, semB, osemA, osemB):
    wid = lax.axis_index("s") * NC + lax.axis_index("c")
    base = wid * EW
    pltpu.sync_copy(idx_hbm.at[pl.ds(base, EW)], ipk)

    def compute_row(row, obuf):
        @plsc.parallel_loop(0, EW, step=L, unroll=16)
        def vec_body(s):
            iv = ipk[pl.ds(s, L)]
            ia = iv & 0xFFFF
            ib = lax.shift_right_logical(iv, 16)
            w1 = plsc.load_gather(row, [ia])
            w2 = plsc.load_gather(row, [ib])
            m1 = plsc.bitcast(w1 & _HI, jnp.float32)
            p1 = plsc.bitcast(lax.shift_left(w1, 16), jnp.float32)
            m2 = plsc.bitcast(w2 & _HI, jnp.float32)
            p2 = plsc.bitcast(lax.shift_left(w2, 16), jnp.float32)
            t1 = m1 * p2
            t2 = m2 * p1
            obuf[pl.ds(s, L)] = (t1 * t2) / (t1 + t2)

    def phase(k, r, rowX, obX, semX, osemX, rowY, semY, pre_r, pre_ok):
        @pl.when(pre_ok)
        def _():
            pltpu.async_copy(mp_hbm.at[pl.ds(pre_r * N, N)], rowY, semY)

        pltpu.make_async_copy(mp_hbm.at[pl.ds(0, N)], rowX, semX).wait()

        @pl.when(k >= 1)
        def _():
            pltpu.make_async_copy(obX, out_hbm.at[pl.ds(0, EW)], osemX).wait()

        compute_row(rowX, obX)
        pltpu.async_copy(obX, out_hbm.at[pl.ds(r * E + base, EW)], osemX)

    pltpu.async_copy(mp_hbm.at[pl.ds(0, N)], rowA, semA)

    def pair_body(k, carry):
        r = 2 * k
        phase(k, r, rowA, obA, semA, osemA, rowB, semB, r + 1, r + 1 < R)
        phase(k, r + 1, rowB, obB, semB, osemB, rowA, semA, r + 2, r + 2 < R)
        return carry

    lax.fori_loop(0, R // 2, pair_body, 0)

    pltpu.make_async_copy(obA, out_hbm.at[pl.ds(0, EW)], osemA).wait()
    pltpu.make_async_copy(obB, out_hbm.at[pl.ds(0, EW)], osemB).wait()


def kernel(m, polar, indices):
    mb = lax.bitcast_convert_type(
        m.astype(jnp.bfloat16), jnp.uint16).astype(jnp.uint32)
    pb = lax.bitcast_convert_type(
        polar.astype(jnp.bfloat16), jnp.uint16).astype(jnp.uint32)
    mp = lax.bitcast_convert_type((mb << 16) | pb, jnp.int32).reshape(-1)
    ipk = indices[0] | (indices[1] << 16)

    mesh = plsc.VectorSubcoreMesh(core_axis_name="c", subcore_axis_name="s")
    f = pl.kernel(
        _body,
        out_type=jax.ShapeDtypeStruct((R * E,), jnp.float32),
        mesh=mesh,
        compiler_params=pltpu.CompilerParams(needs_layout_passes=False),
        scratch_types=[
            pltpu.VMEM((EW,), jnp.int32),
            pltpu.VMEM((N,), jnp.int32),
            pltpu.VMEM((N,), jnp.int32),
            pltpu.VMEM((EW,), jnp.float32),
            pltpu.VMEM((EW,), jnp.float32),
            pltpu.SemaphoreType.DMA,
            pltpu.SemaphoreType.DMA,
            pltpu.SemaphoreType.DMA,
            pltpu.SemaphoreType.DMA,
        ],
    )
    out = f(mp, ipk)
    return out.reshape(R, E)


# speedup vs baseline: 1.0252x; 1.0252x over previous
import jax
import jax.numpy as jnp
from jax import lax
from jax.experimental import pallas as pl
from jax.experimental.pallas import tpu as pltpu
from jax.experimental.pallas import tpu_sc as plsc

R = 128
N = 10000
E = 320000
NC = 2
NS = 16
NW = NC * NS
EW = E // NW
L = 16

_HI = -65536


def _body(mp_hbm, idx_hbm, out_hbm,
          ipk, rowA, rowB, rowC, rowD, ob0, ob1, ob2, ob3,
          semAB, semCD, osem0, osem1, osem2, osem3):
    wid = lax.axis_index("s") * NC + lax.axis_index("c")
    base = wid * EW
    pltpu.sync_copy(idx_hbm.at[pl.ds(base, EW)], ipk)

    def unpack(w):
        mm = plsc.bitcast(w & _HI, jnp.float32)
        pp = plsc.bitcast(lax.shift_left(w, 16), jnp.float32)
        return mm, pp

    def combine(w1, w2):
        m1, p1 = unpack(w1)
        m2, p2 = unpack(w2)
        t1 = m1 * p2
        t2 = m2 * p1
        return (t1 * t2) / (t1 + t2)

    def compute_pair(rowX, rowY, obX, obY):
        @plsc.parallel_loop(0, EW, step=L, unroll=8)
        def vec_body(s):
            iv = ipk[pl.ds(s, L)]
            ia = iv & 0xFFFF
            ib = lax.shift_right_logical(iv, 16)
            w1x = plsc.load_gather(rowX, [ia])
            w2x = plsc.load_gather(rowX, [ib])
            w1y = plsc.load_gather(rowY, [ia])
            w2y = plsc.load_gather(rowY, [ib])
            obX[pl.ds(s, L)] = combine(w1x, w2x)
            obY[pl.ds(s, L)] = combine(w1y, w2y)

    def phase(k, r0, rowX, rowY, obX, obY, semXY, osemX, osemY,
              rowPX, rowPY, semP, pre_r0, pre_ok):
        @pl.when(pre_ok)
        def _():
            pltpu.async_copy(mp_hbm.at[pl.ds(pre_r0 * N, N)], rowPX, semP)
            pltpu.async_copy(mp_hbm.at[pl.ds((pre_r0 + 1) * N, N)], rowPY, semP)

        pltpu.make_async_copy(mp_hbm.at[pl.ds(0, N)], rowX, semXY).wait()
        pltpu.make_async_copy(mp_hbm.at[pl.ds(0, N)], rowY, semXY).wait()

        @pl.when(k >= 1)
        def _():
            pltpu.make_async_copy(obX, out_hbm.at[pl.ds(0, EW)], osemX).wait()
            pltpu.make_async_copy(obY, out_hbm.at[pl.ds(0, EW)], osemY).wait()

        compute_pair(rowX, rowY, obX, obY)
        pltpu.async_copy(obX, out_hbm.at[pl.ds(r0 * E + base, EW)], osemX)
        pltpu.async_copy(obY, out_hbm.at[pl.ds((r0 + 1) * E + base, EW)], osemY)

    pltpu.async_copy(mp_hbm.at[pl.ds(0, N)], rowA, semAB)
    pltpu.async_copy(mp_hbm.at[pl.ds(N, N)], rowB, semAB)

    def quad_body(k, carry):
        r0 = 4 * k
        phase(k, r0, rowA, rowB, ob0, ob1, semAB, osem0, osem1,
              rowC, rowD, semCD, r0 + 2, r0 + 2 < R)
        phase(k, r0 + 2, rowC, rowD, ob2, ob3, semCD, osem2, osem3,
              rowA, rowB, semAB, r0 + 4, r0 + 4 < R)
        return carry

    lax.fori_loop(0, R // 4, quad_body, 0)

    pltpu.make_async_copy(ob0, out_hbm.at[pl.ds(0, EW)], osem0).wait()
    pltpu.make_async_copy(ob1, out_hbm.at[pl.ds(0, EW)], osem1).wait()
    pltpu.make_async_copy(ob2, out_hbm.at[pl.ds(0, EW)], osem2).wait()
    pltpu.make_async_copy(ob3, out_hbm.at[pl.ds(0, EW)], osem3).wait()


def kernel(m, polar, indices):
    mb = lax.bitcast_convert_type(
        m.astype(jnp.bfloat16), jnp.uint16).astype(jnp.uint32)
    pb = lax.bitcast_convert_type(
        polar.astype(jnp.bfloat16), jnp.uint16).astype(jnp.uint32)
    mp = lax.bitcast_convert_type((mb << 16) | pb, jnp.int32).reshape(-1)
    ipk = indices[0] | (indices[1] << 16)

    mesh = plsc.VectorSubcoreMesh(core_axis_name="c", subcore_axis_name="s")
    f = pl.kernel(
        _body,
        out_type=jax.ShapeDtypeStruct((R * E,), jnp.float32),
        mesh=mesh,
        compiler_params=pltpu.CompilerParams(needs_layout_passes=False),
        scratch_types=[
            pltpu.VMEM((EW,), jnp.int32),
            pltpu.VMEM((N,), jnp.int32),
            pltpu.VMEM((N,), jnp.int32),
            pltpu.VMEM((N,), jnp.int32),
            pltpu.VMEM((N,), jnp.int32),
            pltpu.VMEM((EW,), jnp.float32),
            pltpu.VMEM((EW,), jnp.float32),
            pltpu.VMEM((EW,), jnp.float32),
            pltpu.VMEM((EW,), jnp.float32),
            pltpu.SemaphoreType.DMA,
            pltpu.SemaphoreType.DMA,
            pltpu.SemaphoreType.DMA,
            pltpu.SemaphoreType.DMA,
            pltpu.SemaphoreType.DMA,
            pltpu.SemaphoreType.DMA,
        ],
    )
    out = f(mp, ipk)
    return out.reshape(R, E)


# speedup vs baseline: 1.3797x; 1.3457x over previous
import jax
import jax.numpy as jnp
from jax import lax
from jax.experimental import pallas as pl
from jax.experimental.pallas import tpu as pltpu
from jax.experimental.pallas import tpu_sc as plsc

R = 128
N = 10000
E = 320000
NC = 2
NS = 16
NW = NC * NS
L = 16
EWM = 9984
EXTRA = E - NW * EWM
EWX = EWM + EXTRA

_HI = -65536


def _body(mp_hbm, idx_hbm, out_hbm,
          ipk, rowA, rowB, obuf,
          semA, semB, osem, osem2):
    wid = lax.axis_index("s") * NC + lax.axis_index("c")
    base = pl.multiple_of(wid * EWM, 128)
    last = wid == NW - 1
    esz = jnp.where(last, EWX, EWM)

    pltpu.sync_copy(idx_hbm.at[pl.ds(base, EWM)], ipk.at[pl.ds(0, EWM)])

    @pl.when(last)
    def _():
        pltpu.sync_copy(idx_hbm.at[pl.ds(base + EWM, EXTRA)],
                        ipk.at[pl.ds(EWM, EXTRA)])

    def compute_row(row, q):
        @plsc.parallel_loop(0, esz, step=L, unroll=8)
        def vec_body(s):
            iv = ipk[pl.ds(s, L)]
            ia = iv & 0xFFFF
            ib = lax.shift_right_logical(iv, 16)
            w1 = plsc.load_gather(row, [ia])
            w2 = plsc.load_gather(row, [ib])
            m1 = plsc.bitcast(w1 & _HI, jnp.float32)
            p1 = plsc.bitcast(lax.shift_left(w1, 16), jnp.float32)
            m2 = plsc.bitcast(w2 & _HI, jnp.float32)
            p2 = plsc.bitcast(lax.shift_left(w2, 16), jnp.float32)
            t1 = m1 * p2
            t2 = m2 * p1
            obuf[q, pl.ds(s, L)] = (t1 * t2) / (t1 + t2)

    def phase(r, rowX, semX, rowY, semY, pre_r, pre_ok):
        q = r & 7
        b = r >> 3

        @pl.when(pre_ok)
        def _():
            pltpu.async_copy(mp_hbm.at[pl.ds(pre_r * N, N)], rowY, semY)

        @pl.when((q == 0) & (b >= 1))
        def _():
            pltpu.make_async_copy(
                obuf.at[:, pl.ds(0, EWM)],
                out_hbm.at[pl.ds(0, 8), pl.ds(0, EWM)], osem).wait()

        @pl.when((q == 0) & (b >= 1) & last)
        def _():
            pltpu.make_async_copy(
                obuf.at[:, pl.ds(EWM, EXTRA)],
                out_hbm.at[pl.ds(0, 8), pl.ds(0, EXTRA)], osem2).wait()

        pltpu.make_async_copy(mp_hbm.at[pl.ds(0, N)], rowX, semX).wait()

        compute_row(rowX, q)

        @pl.when(q == 7)
        def _():
            r0 = pl.multiple_of(r - 7, 8)
            pltpu.async_copy(
                obuf.at[:, pl.ds(0, EWM)],
                out_hbm.at[pl.ds(r0, 8), pl.ds(base, EWM)], osem)

        @pl.when((q == 7) & last)
        def _():
            r0 = pl.multiple_of(r - 7, 8)
            pltpu.async_copy(
                obuf.at[:, pl.ds(EWM, EXTRA)],
                out_hbm.at[pl.ds(r0, 8), pl.ds(base + EWM, EXTRA)], osem2)

    pltpu.async_copy(mp_hbm.at[pl.ds(0, N)], rowA, semA)

    def pair_body(k, carry):
        r = 2 * k
        phase(r, rowA, semA, rowB, semB, r + 1, r + 1 < R)
        phase(r + 1, rowB, semB, rowA, semA, r + 2, r + 2 < R)
        return carry

    lax.fori_loop(0, R // 2, pair_body, 0)

    pltpu.make_async_copy(
        obuf.at[:, pl.ds(0, EWM)],
        out_hbm.at[pl.ds(0, 8), pl.ds(0, EWM)], osem).wait()

    @pl.when(last)
    def _():
        pltpu.make_async_copy(
            obuf.at[:, pl.ds(EWM, EXTRA)],
            out_hbm.at[pl.ds(0, 8), pl.ds(0, EXTRA)], osem2).wait()


def kernel(m, polar, indices):
    mb = lax.bitcast_convert_type(
        m.astype(jnp.bfloat16), jnp.uint16).astype(jnp.uint32)
    pb = lax.bitcast_convert_type(
        polar.astype(jnp.bfloat16), jnp.uint16).astype(jnp.uint32)
    mp = lax.bitcast_convert_type((mb << 16) | pb, jnp.int32).reshape(-1)
    ipk = indices[0] | (indices[1] << 16)

    mesh = plsc.VectorSubcoreMesh(core_axis_name="c", subcore_axis_name="s")
    f = pl.kernel(
        _body,
        out_type=jax.ShapeDtypeStruct((R, E), jnp.float32),
        mesh=mesh,
        compiler_params=pltpu.CompilerParams(needs_layout_passes=False),
        scratch_types=[
            pltpu.VMEM((EWX,), jnp.int32),
            pltpu.VMEM((N,), jnp.int32),
            pltpu.VMEM((N,), jnp.int32),
            pltpu.VMEM((8, EWX), jnp.float32),
            pltpu.SemaphoreType.DMA,
            pltpu.SemaphoreType.DMA,
            pltpu.SemaphoreType.DMA,
            pltpu.SemaphoreType.DMA,
        ],
    )
    return f(mp, ipk)


# speedup vs baseline: 1.5015x; 1.0883x over previous
import jax
import jax.numpy as jnp
from jax import lax
from jax.experimental import pallas as pl
from jax.experimental.pallas import tpu as pltpu
from jax.experimental.pallas import tpu_sc as plsc

R = 128
N = 10000
E = 320000
NC = 2
NS = 16
NW = NC * NS
L = 16
EWM = 9984
EXTRA = 128
NTAIL = (E - NW * EWM) // EXTRA
EWX = EWM + EXTRA

_HI = -65536


def _body(mp_hbm, idx_hbm, out_hbm,
          ipk, rowA, rowB, obuf,
          semA, semB, osem, osem2):
    wid = lax.axis_index("s") * NC + lax.axis_index("c")
    ext = jnp.maximum(wid - (NW - NTAIL), 0)
    base = pl.multiple_of(wid * EWM + ext * EXTRA, 128)
    last = wid >= NW - NTAIL
    esz = jnp.where(last, EWX, EWM)
    roff = 8 * (wid & 15)

    pltpu.sync_copy(idx_hbm.at[pl.ds(base, EWM)], ipk.at[pl.ds(0, EWM)])

    @pl.when(last)
    def _():
        pltpu.sync_copy(idx_hbm.at[pl.ds(base + EWM, EXTRA)],
                        ipk.at[pl.ds(EWM, EXTRA)])

    def compute_row(row, q):
        @plsc.parallel_loop(0, esz, step=L, unroll=8)
        def vec_body(s):
            iv = ipk[pl.ds(s, L)]
            ia = iv & 0xFFFF
            ib = lax.shift_right_logical(iv, 16)
            w1 = plsc.load_gather(row, [ia])
            w2 = plsc.load_gather(row, [ib])
            m1 = plsc.bitcast(w1 & _HI, jnp.float32)
            p1 = plsc.bitcast(lax.shift_left(w1, 16), jnp.float32)
            m2 = plsc.bitcast(w2 & _HI, jnp.float32)
            p2 = plsc.bitcast(lax.shift_left(w2, 16), jnp.float32)
            t1 = m1 * p2
            t2 = m2 * p1
            obuf[q, pl.ds(s, L)] = (t1 * t2) / (t1 + t2)

    def phase(r, rowX, semX, rowY, semY, pre_r, pre_ok):
        q = r & 7
        b = r >> 3

        @pl.when(pre_ok)
        def _():
            ppre = (pre_r + roff) & 127
            pltpu.async_copy(mp_hbm.at[pl.ds(ppre * N, N)], rowY, semY)

        @pl.when((q == 0) & (b >= 1))
        def _():
            pltpu.make_async_copy(
                obuf.at[:, pl.ds(0, EWM)],
                out_hbm.at[pl.ds(0, 8), pl.ds(0, EWM)], osem).wait()

        @pl.when((q == 0) & (b >= 1) & last)
        def _():
            pltpu.make_async_copy(
                obuf.at[:, pl.ds(EWM, EXTRA)],
                out_hbm.at[pl.ds(0, 8), pl.ds(0, EXTRA)], osem2).wait()

        pltpu.make_async_copy(mp_hbm.at[pl.ds(0, N)], rowX, semX).wait()

        compute_row(rowX, q)

        @pl.when(q == 7)
        def _():
            r0 = pl.multiple_of(((r - 7) + roff) & 127, 8)
            pltpu.async_copy(
                obuf.at[:, pl.ds(0, EWM)],
                out_hbm.at[pl.ds(r0, 8), pl.ds(base, EWM)], osem)

        @pl.when((q == 7) & last)
        def _():
            r0 = pl.multiple_of(((r - 7) + roff) & 127, 8)
            pltpu.async_copy(
                obuf.at[:, pl.ds(EWM, EXTRA)],
                out_hbm.at[pl.ds(r0, 8), pl.ds(base + EWM, EXTRA)], osem2)

    pltpu.async_copy(mp_hbm.at[pl.ds(roff * N, N)], rowA, semA)

    def pair_body(k, carry):
        r = 2 * k
        phase(r, rowA, semA, rowB, semB, r + 1, r + 1 < R)
        phase(r + 1, rowB, semB, rowA, semA, r + 2, r + 2 < R)
        return carry

    lax.fori_loop(0, R // 2, pair_body, 0)

    pltpu.make_async_copy(
        obuf.at[:, pl.ds(0, EWM)],
        out_hbm.at[pl.ds(0, 8), pl.ds(0, EWM)], osem).wait()

    @pl.when(last)
    def _():
        pltpu.make_async_copy(
            obuf.at[:, pl.ds(EWM, EXTRA)],
            out_hbm.at[pl.ds(0, 8), pl.ds(0, EXTRA)], osem2).wait()


def kernel(m, polar, indices):
    mb = lax.bitcast_convert_type(
        m.astype(jnp.bfloat16), jnp.uint16).astype(jnp.uint32)
    pb = lax.bitcast_convert_type(
        polar.astype(jnp.bfloat16), jnp.uint16).astype(jnp.uint32)
    mp = lax.bitcast_convert_type((mb << 16) | pb, jnp.int32).reshape(-1)
    ipk = indices[0] | (indices[1] << 16)

    mesh = plsc.VectorSubcoreMesh(core_axis_name="c", subcore_axis_name="s")
    f = pl.kernel(
        _body,
        out_type=jax.ShapeDtypeStruct((R, E), jnp.float32),
        mesh=mesh,
        compiler_params=pltpu.CompilerParams(needs_layout_passes=False),
        scratch_types=[
            pltpu.VMEM((EWX,), jnp.int32),
            pltpu.VMEM((N,), jnp.int32),
            pltpu.VMEM((N,), jnp.int32),
            pltpu.VMEM((8, EWX), jnp.float32),
            pltpu.SemaphoreType.DMA,
            pltpu.SemaphoreType.DMA,
            pltpu.SemaphoreType.DMA,
            pltpu.SemaphoreType.DMA,
        ],
    )
    return f(mp, ipk)


# speedup vs baseline: 1.6836x; 1.1213x over previous
import jax
import jax.numpy as jnp
from jax import lax
from jax.experimental import pallas as pl
from jax.experimental.pallas import tpu as pltpu
from jax.experimental.pallas import tpu_sc as plsc

R = 128
N = 10000
E = 320000
NC = 2
NS = 16
L = 16
RB = 8
EH = E // NC
CH = 1280
NCH = EH // CH

_HI = -65536


def _body(mp_hbm, idx_hbm, out_hbm,
          rows, ipkA, ipkB, obA, obB,
          isemA, isemB, osemA, osemB):
    grp = lax.axis_index("s")
    half = lax.axis_index("c")
    r0 = pl.multiple_of(grp * RB, 8)
    ebase = pl.multiple_of(half * EH, 128)

    for r in range(RB):
        pltpu.sync_copy(mp_hbm.at[pl.ds((r0 + r) * N, N)],
                        rows.at[pl.ds(r * N, N)])

    def compute_chunk(ipk, ob):
        @plsc.parallel_loop(0, CH, step=L, unroll=4)
        def vec_body(s):
            iv = ipk[pl.ds(s, L)]
            ia = iv & 0xFFFF
            ib = lax.shift_right_logical(iv, 16)
            for r in range(RB):
                row = rows.at[pl.ds(r * N, N)]
                w1 = plsc.load_gather(row, [ia])
                w2 = plsc.load_gather(row, [ib])
                m1 = plsc.bitcast(w1 & _HI, jnp.float32)
                p1 = plsc.bitcast(lax.shift_left(w1, 16), jnp.float32)
                m2 = plsc.bitcast(w2 & _HI, jnp.float32)
                p2 = plsc.bitcast(lax.shift_left(w2, 16), jnp.float32)
                t1 = m1 * p2
                t2 = m2 * p1
                ob[r, pl.ds(s, L)] = (t1 * t2) / (t1 + t2)

    def phase(k, ipkX, isemX, obX, osemX, ipkY, isemY, pre_c, pre_ok):
        @pl.when(pre_ok)
        def _():
            pltpu.async_copy(idx_hbm.at[pl.ds(ebase + pre_c * CH, CH)],
                             ipkY, isemY)

        pltpu.make_async_copy(idx_hbm.at[pl.ds(0, CH)], ipkX, isemX).wait()

        @pl.when(k >= 2)
        def _():
            pltpu.make_async_copy(
                obX, out_hbm.at[pl.ds(0, RB), pl.ds(0, CH)], osemX).wait()

        compute_chunk(ipkX, obX)

        pltpu.async_copy(
            obX,
            out_hbm.at[pl.ds(r0, RB), pl.ds(ebase + k * CH, CH)], osemX)

    pltpu.async_copy(idx_hbm.at[pl.ds(ebase, CH)], ipkA, isemA)

    def pair_body(j, carry):
        k = 2 * j
        phase(k, ipkA, isemA, obA, osemA, ipkB, isemB, k + 1, k + 1 < NCH)
        phase(k + 1, ipkB, isemB, obB, osemB, ipkA, isemA, k + 2, k + 2 < NCH)
        return carry

    lax.fori_loop(0, NCH // 2, pair_body, 0)
    phase(NCH - 1, ipkA, isemA, obA, osemA, ipkB, isemB, 0, False)

    pltpu.make_async_copy(
        obA, out_hbm.at[pl.ds(0, RB), pl.ds(0, CH)], osemA).wait()
    pltpu.make_async_copy(
        obB, out_hbm.at[pl.ds(0, RB), pl.ds(0, CH)], osemB).wait()


def kernel(m, polar, indices):
    mb = lax.bitcast_convert_type(
        m.astype(jnp.bfloat16), jnp.uint16).astype(jnp.uint32)
    pb = lax.bitcast_convert_type(
        polar.astype(jnp.bfloat16), jnp.uint16).astype(jnp.uint32)
    mp = lax.bitcast_convert_type((mb << 16) | pb, jnp.int32).reshape(-1)
    ipk = indices[0] | (indices[1] << 16)

    mesh = plsc.VectorSubcoreMesh(core_axis_name="c", subcore_axis_name="s")
    f = pl.kernel(
        _body,
        out_type=jax.ShapeDtypeStruct((R, E), jnp.float32),
        mesh=mesh,
        compiler_params=pltpu.CompilerParams(needs_layout_passes=False),
        scratch_types=[
            pltpu.VMEM((RB * N,), jnp.int32),
            pltpu.VMEM((CH,), jnp.int32),
            pltpu.VMEM((CH,), jnp.int32),
            pltpu.VMEM((RB, CH), jnp.float32),
            pltpu.VMEM((RB, CH), jnp.float32),
            pltpu.SemaphoreType.DMA,
            pltpu.SemaphoreType.DMA,
            pltpu.SemaphoreType.DMA,
            pltpu.SemaphoreType.DMA,
        ],
    )
    return f(mp, ipk)


# speedup vs baseline: 1.8781x; 1.1156x over previous
import jax
import jax.numpy as jnp
from jax import lax
from jax.experimental import pallas as pl
from jax.experimental.pallas import tpu as pltpu
from jax.experimental.pallas import tpu_sc as plsc

R = 128
N = 10000
E = 320000
NC = 2
NS = 16
L = 16
RB = 8
EH = E // NC
CH = 1280
NCH = EH // CH

_HI = -65536


def _body(mp_hbm, idx_hbm, out_hbm,
          rows, ipkA, ipkB, obA, obB,
          isemA, isemB, osemA, osemB):
    grp = lax.axis_index("s")
    half = lax.axis_index("c")
    r0 = pl.multiple_of(grp * RB, 8)
    ebase = pl.multiple_of(half * EH, 128)

    for r in range(RB):
        pltpu.sync_copy(mp_hbm.at[pl.ds((r0 + r) * N, N)],
                        rows.at[pl.ds(r * N, N)])

    def compute_chunk(ipk, ob):
        @plsc.parallel_loop(0, CH, step=L, unroll=4)
        def vec_body(s):
            iv = ipk[pl.ds(s, L)]
            ia = iv & 0xFFFF
            ib = lax.shift_right_logical(iv, 16)
            for r in range(RB):
                row = rows.at[pl.ds(r * N, N)]
                w1 = plsc.load_gather(row, [ia])
                w2 = plsc.load_gather(row, [ib])
                m1 = plsc.bitcast(w1, jnp.float32)
                p1 = plsc.bitcast(lax.shift_left(w1, 16), jnp.float32)
                m2 = plsc.bitcast(w2, jnp.float32)
                p2 = plsc.bitcast(lax.shift_left(w2, 16), jnp.float32)
                t1 = m1 * p2
                t2 = m2 * p1
                ob[r, pl.ds(s, L)] = (t1 * t2) / (t1 + t2)

    def phase(k, ipkX, isemX, obX, osemX, ipkY, isemY, pre_c, pre_ok):
        @pl.when(pre_ok)
        def _():
            pltpu.async_copy(idx_hbm.at[pl.ds(ebase + pre_c * CH, CH)],
                             ipkY, isemY)

        pltpu.make_async_copy(idx_hbm.at[pl.ds(0, CH)], ipkX, isemX).wait()

        @pl.when(k >= 2)
        def _():
            pltpu.make_async_copy(
                obX, out_hbm.at[pl.ds(0, RB), pl.ds(0, CH)], osemX).wait()

        compute_chunk(ipkX, obX)

        pltpu.async_copy(
            obX,
            out_hbm.at[pl.ds(r0, RB), pl.ds(ebase + k * CH, CH)], osemX)

    pltpu.async_copy(idx_hbm.at[pl.ds(ebase, CH)], ipkA, isemA)

    def pair_body(j, carry):
        k = 2 * j
        phase(k, ipkA, isemA, obA, osemA, ipkB, isemB, k + 1, k + 1 < NCH)
        phase(k + 1, ipkB, isemB, obB, osemB, ipkA, isemA, k + 2, k + 2 < NCH)
        return carry

    lax.fori_loop(0, NCH // 2, pair_body, 0)
    phase(NCH - 1, ipkA, isemA, obA, osemA, ipkB, isemB, 0, False)

    pltpu.make_async_copy(
        obA, out_hbm.at[pl.ds(0, RB), pl.ds(0, CH)], osemA).wait()
    pltpu.make_async_copy(
        obB, out_hbm.at[pl.ds(0, RB), pl.ds(0, CH)], osemB).wait()


def kernel(m, polar, indices):
    mb = lax.bitcast_convert_type(
        m.astype(jnp.bfloat16), jnp.uint16).astype(jnp.uint32)
    pb = lax.bitcast_convert_type(
        polar.astype(jnp.bfloat16), jnp.uint16).astype(jnp.uint32)
    mp = lax.bitcast_convert_type((mb << 16) | pb, jnp.int32).reshape(-1)
    ipk = indices[0] | (indices[1] << 16)

    mesh = plsc.VectorSubcoreMesh(core_axis_name="c", subcore_axis_name="s")
    f = pl.kernel(
        _body,
        out_type=jax.ShapeDtypeStruct((R, E), jnp.float32),
        mesh=mesh,
        compiler_params=pltpu.CompilerParams(needs_layout_passes=False),
        scratch_types=[
            pltpu.VMEM((RB * N,), jnp.int32),
            pltpu.VMEM((CH,), jnp.int32),
            pltpu.VMEM((CH,), jnp.int32),
            pltpu.VMEM((RB, CH), jnp.float32),
            pltpu.VMEM((RB, CH), jnp.float32),
            pltpu.SemaphoreType.DMA,
            pltpu.SemaphoreType.DMA,
            pltpu.SemaphoreType.DMA,
            pltpu.SemaphoreType.DMA,
        ],
    )
    return f(mp, ipk)
